# baseline (speedup 1.0000x reference)
import functools

import jax
import jax.numpy as jnp
import numpy as np
from jax import lax
from jax.experimental import pallas as pl
from jax.experimental.pallas import tpu as pltpu
from jax.experimental.pallas import tpu_sc as plsc

B = 16384
C = 1000
SMOOTHING = 0.1
CONF = 1.0 - SMOOTHING
OFF = SMOOTHING / (C - 1)
KLSE = CONF - OFF + OFF * C
ENT = CONF * float(np.log(CONF)) + (C - 1) * OFF * float(np.log(OFF))

B_SC = 4096
B_TC = B - B_SC
COLS_PER_BLOCK = 2048


def _dense_body(x_ref, lab_ref, acc_ref):
    i = pl.program_id(0)
    x = x_ref[...]
    lab = lab_ref[0, 0, :]
    m = jnp.max(x, axis=0, keepdims=True)
    lse = m + jnp.log(jnp.sum(jnp.exp(x - m), axis=0, keepdims=True))
    row = lax.broadcasted_iota(jnp.int32, (C, COLS_PER_BLOCK), 0)
    g = jnp.sum(jnp.where(row == lab[None, :], x, 0.0))
    partial = jnp.reshape(
        OFF * jnp.sum(x) - KLSE * jnp.sum(lse) + (CONF - OFF) * g, (1, 1)
    )

    @pl.when(i == 0)
    def _():
        acc_ref[...] = jnp.zeros((1, 1), jnp.float32)

    acc_ref[...] += partial


_dense_call = pl.pallas_call(
    _dense_body,
    grid=(B_TC // COLS_PER_BLOCK,),
    in_specs=[
        pl.BlockSpec((C, COLS_PER_BLOCK), lambda i: (0, i)),
        pl.BlockSpec((1, 1, COLS_PER_BLOCK), lambda i: (i, 0, 0)),
    ],
    out_specs=pl.BlockSpec((1, 1), lambda i: (0, 0)),
    out_shape=jax.ShapeDtypeStruct((1, 1), jnp.float32),
)


_INFO = plsc.get_sparse_core_info()
_NC = _INFO.num_cores
_NS = _INFO.num_subcores
_NW = _NC * _NS
_CPT = B_SC // _NW
_NCG = _CPT // 16
_RS = 200
_NSL = C // _RS

_sc_mesh = plsc.VectorSubcoreMesh(core_axis_name="c", subcore_axis_name="s")


@functools.partial(
    pl.kernel,
    out_type=jax.ShapeDtypeStruct((4, B_SC), jnp.float32),
    mesh=_sc_mesh,
    compiler_params=pltpu.CompilerParams(use_tc_tiling_on_sc=True),
    scratch_types=[
        pltpu.VMEM((_CPT,), jnp.int32),
        pltpu.VMEM((_RS, _CPT), jnp.float32),
        pltpu.VMEM((4, _CPT), jnp.float32),
    ],
)
def _sc_dense(xt_hbm, labels_hbm, out_hbm, lab_v, xbuf, st_v):
    cc = lax.axis_index("c")
    ss = lax.axis_index("s")
    wid = ss * _NC + cc
    col0 = B_TC + wid * _CPT
    pltpu.sync_copy(labels_hbm.at[pl.ds(col0, _CPT)], lab_v)
    labs = [lab_v[pl.ds(cg * 16, 16)] for cg in range(_NCG)]
    neg = jnp.full((16,), -jnp.inf, jnp.float32)
    zero = jnp.zeros((16,), jnp.float32)
    m = [neg] * _NCG
    s = [zero] * _NCG
    rs = [zero] * _NCG
    g = [zero] * _NCG
    for sl in range(_NSL):
        pltpu.sync_copy(
            xt_hbm.at[pl.ds(sl * _RS, _RS), pl.ds(col0, _CPT)], xbuf
        )

        def p1(r, carry):
            return tuple(
                jnp.maximum(carry[cg], xbuf[r, pl.ds(cg * 16, 16)])
                for cg in range(_NCG)
            )

        msl = lax.fori_loop(0, _RS, p1, tuple([neg] * _NCG))
        mnew = [jnp.maximum(m[cg], msl[cg]) for cg in range(_NCG)]
        s = [s[cg] * jnp.exp(m[cg] - mnew[cg]) for cg in range(_NCG)]
        m = mnew

        def p2(r, carry):
            cs, crs, cg_ = carry
            rowv = jnp.full((16,), sl * _RS, jnp.int32) + r
            ns, nrs, ng = [], [], []
            for cg in range(_NCG):
                x = xbuf[r, pl.ds(cg * 16, 16)]
                ns.append(cs[cg] + jnp.exp(x - m[cg]))
                nrs.append(crs[cg] + x)
                ng.append(cg_[cg] + jnp.where(rowv == labs[cg], x, 0.0))
            return tuple(ns), tuple(nrs), tuple(ng)

        s, rs, g = lax.fori_loop(0, _RS, p2, (tuple(s), tuple(rs), tuple(g)))
        s, rs, g = list(s), list(rs), list(g)
    for cg in range(_NCG):
        st_v[0, pl.ds(cg * 16, 16)] = m[cg]
        st_v[1, pl.ds(cg * 16, 16)] = s[cg]
        st_v[2, pl.ds(cg * 16, 16)] = rs[cg]
        st_v[3, pl.ds(cg * 16, 16)] = g[cg]
    pltpu.sync_copy(st_v, out_hbm.at[:, pl.ds(col0 - B_TC, _CPT)])


def _finish_body(st_ref, acc_ref):
    m = st_ref[0, :]
    s = st_ref[1, :]
    rs = st_ref[2, :]
    g = st_ref[3, :]
    lse = m + jnp.log(s)
    acc_ref[...] = jnp.reshape(
        OFF * jnp.sum(rs) - KLSE * jnp.sum(lse) + (CONF - OFF) * jnp.sum(g),
        (1, 1),
    )


_finish_call = pl.pallas_call(
    _finish_body,
    out_shape=jax.ShapeDtypeStruct((1, 1), jnp.float32),
)


@jax.jit
def kernel(outputs, labels):
    xt = jnp.transpose(outputs)
    lab32 = labels.astype(jnp.int32)
    stats = _sc_dense(xt, lab32)
    lab3 = jnp.reshape(lab32, (B // COLS_PER_BLOCK, 1, COLS_PER_BLOCK))
    acc_tc = _dense_call(xt, lab3)[0, 0]
    acc_sc = _finish_call(stats)[0, 0]
    return ENT - (acc_tc + acc_sc) / B


# speedup vs baseline: 1.1235x; 1.1235x over previous
import functools

import jax
import jax.numpy as jnp
import numpy as np
from jax import lax
from jax.experimental import pallas as pl
from jax.experimental.pallas import tpu as pltpu
from jax.experimental.pallas import tpu_sc as plsc

B = 16384
C = 1000
SMOOTHING = 0.1
CONF = 1.0 - SMOOTHING
OFF = SMOOTHING / (C - 1)
KLSE = CONF - OFF + OFF * C
ENT = CONF * float(np.log(CONF)) + (C - 1) * OFF * float(np.log(OFF))

B_SC = 4096
B_TC = B - B_SC
COLS_PER_BLOCK = 2048


def _dense_body(x_ref, lab_ref, acc_ref):
    i = pl.program_id(0)
    x = x_ref[...]
    lab = lab_ref[0, 0, :]
    m = jnp.max(x, axis=0, keepdims=True)
    lse = m + jnp.log(jnp.sum(jnp.exp(x - m), axis=0, keepdims=True))
    row = lax.broadcasted_iota(jnp.int32, (C, COLS_PER_BLOCK), 0)
    g = jnp.sum(jnp.where(row == lab[None, :], x, 0.0))
    partial = jnp.reshape(
        OFF * jnp.sum(x) - KLSE * jnp.sum(lse) + (CONF - OFF) * g, (1, 1)
    )

    @pl.when(i == 0)
    def _():
        acc_ref[...] = jnp.zeros((1, 1), jnp.float32)

    acc_ref[...] += partial


_dense_call = pl.pallas_call(
    _dense_body,
    grid=(B_TC // COLS_PER_BLOCK,),
    in_specs=[
        pl.BlockSpec((C, COLS_PER_BLOCK), lambda i: (0, i)),
        pl.BlockSpec((1, 1, COLS_PER_BLOCK), lambda i: (i, 0, 0)),
    ],
    out_specs=pl.BlockSpec((1, 1), lambda i: (0, 0)),
    out_shape=jax.ShapeDtypeStruct((1, 1), jnp.float32),
)


_INFO = plsc.get_sparse_core_info()
_NC = _INFO.num_cores
_NS = _INFO.num_subcores
_NW = _NC * _NS
_CPT = B_SC // _NW
_NCG = _CPT // 16
_RS = 200
_NSL = C // _RS

_sc_mesh = plsc.VectorSubcoreMesh(core_axis_name="c", subcore_axis_name="s")


_LN2 = 0.6931471805599453


def _ln(s):
    bits = lax.bitcast_convert_type(s, jnp.int32)
    k = lax.shift_right_arithmetic(bits, 23) - 127
    f = lax.bitcast_convert_type(
        lax.bitwise_or(lax.bitwise_and(bits, 0x7FFFFF), 0x3F800000), jnp.float32
    )
    y = _LN2 * k.astype(jnp.float32) + 2.0 * (f - 1.0) / (f + 1.0)
    y = y - 1.0 + s * jnp.exp(-y)
    y = y - 1.0 + s * jnp.exp(-y)
    return y


@functools.partial(
    pl.kernel,
    out_type=jax.ShapeDtypeStruct((_NW, 16), jnp.float32),
    mesh=_sc_mesh,
    compiler_params=pltpu.CompilerParams(use_tc_tiling_on_sc=True),
    scratch_types=[
        pltpu.VMEM((_CPT,), jnp.int32),
        pltpu.VMEM((2, _RS, _CPT), jnp.float32),
        pltpu.VMEM((16,), jnp.float32),
        pltpu.SemaphoreType.DMA,
        pltpu.SemaphoreType.DMA,
    ],
)
def _sc_dense(xt_hbm, labels_hbm, out_hbm, lab_v, xbuf, st_v, sem0, sem1):
    cc = lax.axis_index("c")
    ss = lax.axis_index("s")
    wid = ss * _NC + cc
    col0 = B_TC + wid * _CPT
    pltpu.sync_copy(labels_hbm.at[pl.ds(col0, _CPT)], lab_v)
    labs = [lab_v[pl.ds(cg * 16, 16)] for cg in range(_NCG)]
    neg = jnp.full((16,), -jnp.inf, jnp.float32)
    zero = jnp.zeros((16,), jnp.float32)
    m = [neg] * _NCG
    s = [zero] * _NCG
    rs = [zero] * _NCG
    g = [zero] * _NCG
    sems = (sem0, sem1)
    copies = [None] * _NSL
    copies[0] = pltpu.async_copy(
        xt_hbm.at[pl.ds(0, _RS), pl.ds(col0, _CPT)], xbuf.at[0], sems[0]
    )
    for sl in range(_NSL):
        b = sl % 2
        copies[sl].wait()
        if sl + 1 < _NSL:
            copies[sl + 1] = pltpu.async_copy(
                xt_hbm.at[pl.ds((sl + 1) * _RS, _RS), pl.ds(col0, _CPT)],
                xbuf.at[1 - b],
                sems[1 - b],
            )

        def p1(r, carry, b=b):
            return tuple(
                jnp.maximum(carry[cg], xbuf[b, r, pl.ds(cg * 16, 16)])
                for cg in range(_NCG)
            )

        msl = lax.fori_loop(0, _RS, p1, tuple([neg] * _NCG))
        mnew = [jnp.maximum(m[cg], msl[cg]) for cg in range(_NCG)]
        s = [s[cg] * jnp.exp(m[cg] - mnew[cg]) for cg in range(_NCG)]
        m = mnew

        def p2(r, carry, b=b, sl=sl):
            cs, crs, cg_ = carry
            rowv = jnp.full((16,), sl * _RS, jnp.int32) + r
            ns, nrs, ng = [], [], []
            for cg in range(_NCG):
                x = xbuf[b, r, pl.ds(cg * 16, 16)]
                ns.append(cs[cg] + jnp.exp(x - m[cg]))
                nrs.append(crs[cg] + x)
                ng.append(cg_[cg] + jnp.where(rowv == labs[cg], x, 0.0))
            return tuple(ns), tuple(nrs), tuple(ng)

        s, rs, g = lax.fori_loop(0, _RS, p2, (tuple(s), tuple(rs), tuple(g)))
        s, rs, g = list(s), list(rs), list(g)
    part = zero
    for cg in range(_NCG):
        lse = m[cg] + _ln(s[cg])
        part = part + (OFF * rs[cg] - KLSE * lse + (CONF - OFF) * g[cg])
    st_v[...] = part
    pltpu.sync_copy(st_v, out_hbm.at[wid])


@jax.jit
def kernel(outputs, labels):
    xt = jnp.transpose(outputs)
    lab32 = labels.astype(jnp.int32)
    parts = _sc_dense(xt, lab32)
    lab3 = jnp.reshape(lab32, (B // COLS_PER_BLOCK, 1, COLS_PER_BLOCK))
    acc_tc = _dense_call(xt, lab3)[0, 0]
    return ENT - (acc_tc + jnp.sum(parts)) / B
